# baseline (speedup 1.0000x reference)
import functools

import jax
import jax.numpy as jnp
from jax.experimental import pallas as pl
from jax.experimental.pallas import tpu as pltpu

_BM = 400


def _u1_kernel(x_ref, w_ref, b_ref, o_ref):
    u = jnp.dot(
        x_ref[...].astype(jnp.bfloat16),
        w_ref[...].astype(jnp.bfloat16),
        preferred_element_type=jnp.float32,
    ) + b_ref[...]
    o_ref[...] = u.astype(jnp.bfloat16)


def _l1_kernel(adj_ref, u1_ref, w2_ref, b2_ref, z_ref):
    h = jnp.dot(
        adj_ref[...].astype(jnp.bfloat16),
        u1_ref[...],
        preferred_element_type=jnp.float32,
    )
    h = jnp.maximum(h, 0.0).astype(jnp.bfloat16)
    z = jnp.dot(h, w2_ref[...], preferred_element_type=jnp.float32) + b2_ref[...]
    z_ref[...] = z.astype(jnp.bfloat16)


def _l2_kernel(adj_ref, z_ref, o_ref):
    o_ref[...] = jnp.dot(
        adj_ref[...].astype(jnp.bfloat16),
        z_ref[...],
        preferred_element_type=jnp.float32,
    )


@functools.partial(jax.jit, static_argnames=())
def kernel(x, Adj, W1, b1, W2, b2):
    n, d_in = x.shape
    d_hid = W1.shape[1]
    d_out = W2.shape[1]

    u1 = pl.pallas_call(
        _u1_kernel,
        grid=(n // 1000,),
        in_specs=[
            pl.BlockSpec((1000, d_in), lambda i: (i, 0)),
            pl.BlockSpec((d_in, d_hid), lambda i: (0, 0)),
            pl.BlockSpec((1, d_hid), lambda i: (0, 0)),
        ],
        out_specs=pl.BlockSpec((1000, d_hid), lambda i: (i, 0)),
        out_shape=jax.ShapeDtypeStruct((n, d_hid), jnp.bfloat16),
    )(x, W1, b1.reshape(1, -1))

    z = pl.pallas_call(
        _l1_kernel,
        grid=(n // _BM,),
        in_specs=[
            pl.BlockSpec((_BM, n), lambda i: (i, 0)),
            pl.BlockSpec((n, d_hid), lambda i: (0, 0)),
            pl.BlockSpec((d_hid, d_out), lambda i: (0, 0)),
            pl.BlockSpec((1, d_out), lambda i: (0, 0)),
        ],
        out_specs=pl.BlockSpec((_BM, d_out), lambda i: (i, 0)),
        out_shape=jax.ShapeDtypeStruct((n, d_out), jnp.bfloat16),
        compiler_params=pltpu.CompilerParams(
            dimension_semantics=("arbitrary",),
        ),
    )(Adj, u1, W2.astype(jnp.bfloat16), b2.reshape(1, -1))

    out = pl.pallas_call(
        _l2_kernel,
        grid=(n // _BM,),
        in_specs=[
            pl.BlockSpec((_BM, n), lambda i: (i, 0)),
            pl.BlockSpec((n, d_out), lambda i: (0, 0)),
        ],
        out_specs=pl.BlockSpec((_BM, d_out), lambda i: (i, 0)),
        out_shape=jax.ShapeDtypeStruct((n, d_out), jnp.float32),
        compiler_params=pltpu.CompilerParams(
            dimension_semantics=("arbitrary",),
        ),
    )(Adj, z)

    return out
